# baseline (speedup 1.0000x reference)
import jax
import jax.numpy as jnp
from jax import lax
from jax.experimental import pallas as pl
from jax.experimental.pallas import tpu as pltpu
from jax.experimental.pallas import tpu_sc as plsc

NUM_EXPERTS = 64
K = 8
TOKENS = 16384
LANES = 16
NUM_WORKERS = 32
ROW_LANES = 128
NUM_ROWS = TOKENS * NUM_EXPERTS // ROW_LANES
ROWS_PER_WORKER = NUM_ROWS // NUM_WORKERS
CHUNK_ROWS = 128
CHUNKS = ((0, 64), (64, 128), (192, 64))
NUM_CHUNKS = len(CHUNKS)


def _sortd(x):
    sk, _ = plsc.sort_key_val(x, x, descending=True)
    return sk


def _sorta(x):
    sk, _ = plsc.sort_key_val(x, x, descending=False)
    return sk


def _kth_of_token(v0, v1, v2, v3):
    w01 = jnp.maximum(_sortd(v0), _sorta(v1))
    w23 = jnp.maximum(_sortd(v2), _sorta(v3))
    f = jnp.maximum(_sortd(w01), _sorta(w23))
    fs = _sortd(f)
    idx7 = jnp.full((LANES,), K - 1, jnp.int32)
    return fs.at[idx7].get(mode="promise_in_bounds")


def _body(scores_hbm, mask_hbm, gated_hbm,
          in_v, mask_v, gated_v, in_sems, out_sems):
    wid = lax.axis_index("s") * 2 + lax.axis_index("c")
    base = wid * ROWS_PER_WORKER

    def in_copy(c, buf):
        start, rows = CHUNKS[c]
        return pltpu.make_async_copy(
            scores_hbm.at[pl.ds(base + start, rows)],
            in_v.at[buf, pl.ds(0, rows)], in_sems.at[buf])

    def out_copies(c, buf):
        start, rows = CHUNKS[c]
        return (
            pltpu.make_async_copy(
                mask_v.at[buf, pl.ds(0, rows)],
                mask_hbm.at[pl.ds(base + start, rows)],
                out_sems.at[0, buf]),
            pltpu.make_async_copy(
                gated_v.at[buf, pl.ds(0, rows)],
                gated_hbm.at[pl.ds(base + start, rows)],
                out_sems.at[1, buf]),
        )

    in_copy(0, 0).start()
    in_copy(1, 1).start()
    for c in range(NUM_CHUNKS):
        buf = c % 2
        in_copy(c, buf).wait()
        if c >= 2:
            for cp in out_copies(c - 2, buf):
                cp.wait()

        @plsc.parallel_loop(0, CHUNKS[c][1], step=1, unroll=2)
        def _row(r):
            for tok in range(2):
                off = tok * NUM_EXPERTS
                v0 = in_v[buf, r, pl.ds(off, LANES)]
                v1 = in_v[buf, r, pl.ds(off + LANES, LANES)]
                v2 = in_v[buf, r, pl.ds(off + 2 * LANES, LANES)]
                v3 = in_v[buf, r, pl.ds(off + 3 * LANES, LANES)]
                tau = _kth_of_token(v0, v1, v2, v3)
                for j, v in enumerate((v0, v1, v2, v3)):
                    m = jnp.where(v >= tau, 1.0, 0.0).astype(jnp.float32)
                    mask_v[buf, r, pl.ds(off + j * LANES, LANES)] = m
                    gated_v[buf, r, pl.ds(off + j * LANES, LANES)] = v * m

        if c + 2 < NUM_CHUNKS:
            in_copy(c + 2, buf).start()
        for cp in out_copies(c, buf):
            cp.start()

    for c in (NUM_CHUNKS - 2, NUM_CHUNKS - 1):
        for cp in out_copies(c, c % 2):
            cp.wait()


@jax.jit
def kernel(routing_tensor):
    packed = routing_tensor.reshape(NUM_ROWS, ROW_LANES)
    out_sds = jax.ShapeDtypeStruct((NUM_ROWS, ROW_LANES), jnp.float32)
    buf = pltpu.VMEM((2, CHUNK_ROWS, ROW_LANES), jnp.float32)
    run = pl.kernel(
        _body,
        out_type=(out_sds, out_sds),
        mesh=plsc.VectorSubcoreMesh(
            core_axis_name="c", subcore_axis_name="s",
            num_cores=2, num_subcores=16,
        ),
        scratch_types=[buf, buf, buf,
                       pltpu.SemaphoreType.DMA((2,)),
                       pltpu.SemaphoreType.DMA((2, 2))],
        compiler_params=pltpu.CompilerParams(needs_layout_passes=False),
    )
    mask_p, gated_p = run(packed)
    shape = (TOKENS, NUM_EXPERTS)
    return mask_p.reshape(shape), gated_p.reshape(shape)


# speedup vs baseline: 1.0007x; 1.0007x over previous
import jax
import jax.numpy as jnp
from jax import lax
from jax.experimental import pallas as pl
from jax.experimental.pallas import tpu as pltpu
from jax.experimental.pallas import tpu_sc as plsc

NUM_EXPERTS = 64
K = 8
TOKENS = 16384
LANES = 16
NUM_WORKERS = 32
ROW_LANES = 128
NUM_ROWS = TOKENS * NUM_EXPERTS // ROW_LANES
ROWS_PER_WORKER = NUM_ROWS // NUM_WORKERS
CHUNK_ROWS = 128
CHUNKS = ((0, 128), (128, 128))
NUM_CHUNKS = len(CHUNKS)


def _sortd(x):
    sk, _ = plsc.sort_key_val(x, x, descending=True)
    return sk


def _sorta(x):
    sk, _ = plsc.sort_key_val(x, x, descending=False)
    return sk


def _kth_of_token(v0, v1, v2, v3):
    w01 = jnp.maximum(_sortd(v0), _sorta(v1))
    w23 = jnp.maximum(_sortd(v2), _sorta(v3))
    f = jnp.maximum(_sortd(w01), _sorta(w23))
    fs = _sortd(f)
    idx7 = jnp.full((LANES,), K - 1, jnp.int32)
    return fs.at[idx7].get(mode="promise_in_bounds")


def _body(scores_hbm, mask_hbm, gated_hbm,
          in_v, mask_v, gated_v, in_sems, out_sems):
    wid = lax.axis_index("s") * 2 + lax.axis_index("c")
    base = wid * ROWS_PER_WORKER

    def in_copy(c, buf):
        start, rows = CHUNKS[c]
        return pltpu.make_async_copy(
            scores_hbm.at[pl.ds(base + start, rows)],
            in_v.at[buf, pl.ds(0, rows)], in_sems.at[buf])

    def out_copies(c, buf):
        start, rows = CHUNKS[c]
        return (
            pltpu.make_async_copy(
                mask_v.at[buf, pl.ds(0, rows)],
                mask_hbm.at[pl.ds(base + start, rows)],
                out_sems.at[0, buf]),
            pltpu.make_async_copy(
                gated_v.at[buf, pl.ds(0, rows)],
                gated_hbm.at[pl.ds(base + start, rows)],
                out_sems.at[1, buf]),
        )

    in_copy(0, 0).start()
    in_copy(1, 1).start()
    for c in range(NUM_CHUNKS):
        buf = c % 2
        in_copy(c, buf).wait()
        if c >= 2:
            for cp in out_copies(c - 2, buf):
                cp.wait()

        @plsc.parallel_loop(0, CHUNKS[c][1], step=1, unroll=2)
        def _row(r):
            for tok in range(2):
                off = tok * NUM_EXPERTS
                v0 = in_v[buf, r, pl.ds(off, LANES)]
                v1 = in_v[buf, r, pl.ds(off + LANES, LANES)]
                v2 = in_v[buf, r, pl.ds(off + 2 * LANES, LANES)]
                v3 = in_v[buf, r, pl.ds(off + 3 * LANES, LANES)]
                tau = _kth_of_token(v0, v1, v2, v3)
                for j, v in enumerate((v0, v1, v2, v3)):
                    m = jnp.where(v >= tau, 1.0, 0.0).astype(jnp.float32)
                    mask_v[buf, r, pl.ds(off + j * LANES, LANES)] = m
                    gated_v[buf, r, pl.ds(off + j * LANES, LANES)] = v * m

        if c + 2 < NUM_CHUNKS:
            in_copy(c + 2, buf).start()
        for cp in out_copies(c, buf):
            cp.start()

    for c in (NUM_CHUNKS - 2, NUM_CHUNKS - 1):
        for cp in out_copies(c, c % 2):
            cp.wait()


@jax.jit
def kernel(routing_tensor):
    packed = routing_tensor.reshape(NUM_ROWS, ROW_LANES)
    out_sds = jax.ShapeDtypeStruct((NUM_ROWS, ROW_LANES), jnp.float32)
    buf = pltpu.VMEM((2, CHUNK_ROWS, ROW_LANES), jnp.float32)
    run = pl.kernel(
        _body,
        out_type=(out_sds, out_sds),
        mesh=plsc.VectorSubcoreMesh(
            core_axis_name="c", subcore_axis_name="s",
            num_cores=2, num_subcores=16,
        ),
        scratch_types=[buf, buf, buf,
                       pltpu.SemaphoreType.DMA((2,)),
                       pltpu.SemaphoreType.DMA((2, 2))],
        compiler_params=pltpu.CompilerParams(needs_layout_passes=False),
    )
    mask_p, gated_p = run(packed)
    shape = (TOKENS, NUM_EXPERTS)
    return mask_p.reshape(shape), gated_p.reshape(shape)


# speedup vs baseline: 1.0081x; 1.0074x over previous
import jax
import jax.numpy as jnp
from jax import lax
from jax.experimental import pallas as pl
from jax.experimental.pallas import tpu as pltpu
from jax.experimental.pallas import tpu_sc as plsc

NUM_EXPERTS = 64
K = 8
TOKENS = 16384
LANES = 16
NUM_WORKERS = 32
ROW_LANES = 128
NUM_ROWS = TOKENS * NUM_EXPERTS // ROW_LANES
ROWS_PER_WORKER = NUM_ROWS // NUM_WORKERS
CHUNK_ROWS = 128
CHUNKS = ((0, 128), (128, 128))
NUM_CHUNKS = len(CHUNKS)


def _sortd(x):
    sk, _ = plsc.sort_key_val(x, x, descending=True)
    return sk


def _sorta(x):
    sk, _ = plsc.sort_key_val(x, x, descending=False)
    return sk


def _kth_of_token(v0, v1, v2, v3):
    w01 = jnp.maximum(_sortd(v0), _sorta(v1))
    w23 = jnp.maximum(_sortd(v2), _sorta(v3))
    f = jnp.maximum(_sortd(w01), _sorta(w23))
    fs = _sortd(f)
    idx7 = jnp.full((LANES,), K - 1, jnp.int32)
    return fs.at[idx7].get(mode="promise_in_bounds")


def _body(scores_hbm, mask_hbm, gated_hbm,
          in_v, mask_v, gated_v, in_sems, out_sems):
    wid = lax.axis_index("s") * 2 + lax.axis_index("c")
    base = wid * ROWS_PER_WORKER

    def in_copy(c, buf):
        start, rows = CHUNKS[c]
        return pltpu.make_async_copy(
            scores_hbm.at[pl.ds(base + start, rows)],
            in_v.at[buf, pl.ds(0, rows)], in_sems.at[buf])

    def out_copies(c, buf):
        start, rows = CHUNKS[c]
        return (
            pltpu.make_async_copy(
                mask_v.at[buf, pl.ds(0, rows)],
                mask_hbm.at[pl.ds(base + start, rows)],
                out_sems.at[0, buf]),
            pltpu.make_async_copy(
                gated_v.at[buf, pl.ds(0, rows)],
                gated_hbm.at[pl.ds(base + start, rows)],
                out_sems.at[1, buf]),
        )

    in_copy(0, 0).start()
    for c in range(NUM_CHUNKS):
        buf = c % 2
        in_copy(c, buf).wait()
        if c + 1 < NUM_CHUNKS:
            in_copy(c + 1, 1 - buf).start()
        if c >= 2:
            for cp in out_copies(c - 2, buf):
                cp.wait()

        @plsc.parallel_loop(0, CHUNKS[c][1], step=1, unroll=2)
        def _row(r):
            for tok in range(2):
                off = tok * NUM_EXPERTS
                v0 = in_v[buf, r, pl.ds(off, LANES)]
                v1 = in_v[buf, r, pl.ds(off + LANES, LANES)]
                v2 = in_v[buf, r, pl.ds(off + 2 * LANES, LANES)]
                v3 = in_v[buf, r, pl.ds(off + 3 * LANES, LANES)]
                tau = _kth_of_token(v0, v1, v2, v3)
                for j, v in enumerate((v0, v1, v2, v3)):
                    m = jnp.where(v >= tau, 1.0, 0.0).astype(jnp.float32)
                    mask_v[buf, r, pl.ds(off + j * LANES, LANES)] = m
                    gated_v[buf, r, pl.ds(off + j * LANES, LANES)] = v * m

        for cp in out_copies(c, buf):
            cp.start()

    for c in (NUM_CHUNKS - 2, NUM_CHUNKS - 1):
        for cp in out_copies(c, c % 2):
            cp.wait()


@jax.jit
def kernel(routing_tensor):
    packed = routing_tensor.reshape(NUM_ROWS, ROW_LANES)
    out_sds = jax.ShapeDtypeStruct((NUM_ROWS, ROW_LANES), jnp.float32)
    buf = pltpu.VMEM((2, CHUNK_ROWS, ROW_LANES), jnp.float32)
    run = pl.kernel(
        _body,
        out_type=(out_sds, out_sds),
        mesh=plsc.VectorSubcoreMesh(
            core_axis_name="c", subcore_axis_name="s",
            num_cores=2, num_subcores=16,
        ),
        scratch_types=[buf, buf, buf,
                       pltpu.SemaphoreType.DMA((2,)),
                       pltpu.SemaphoreType.DMA((2, 2))],
        compiler_params=pltpu.CompilerParams(needs_layout_passes=False),
    )
    mask_p, gated_p = run(packed)
    shape = (TOKENS, NUM_EXPERTS)
    return mask_p.reshape(shape), gated_p.reshape(shape)
